# baseline (speedup 1.0000x reference)
import functools

import jax
import jax.numpy as jnp
from jax import lax
from jax.experimental import pallas as pl
from jax.experimental.pallas import tpu as pltpu
from jax.experimental.pallas import tpu_sc as plsc

N_NODES = 10000
N_EDGES = 320000
D = 128

NC = 2
NS = 16
NW = NC * NS
E_PER_W = N_EDGES // NW
CHUNK = 40
N_CHUNKS = E_PER_W // CHUNK
N_PAD = 10240
ROWS_PER_TILE = N_PAD // NS
ZCHUNK = 64

BE = 2560
BN = 2000


def _mm_bias_body(a_ref, w_ref, b_ref, o_ref):
    o_ref[...] = (
        jnp.dot(a_ref[...], w_ref[...], preferred_element_type=jnp.float32)
        + b_ref[...]
    )


def _edge_proj(edge_attr, lin_Wt, lin_b):
    return pl.pallas_call(
        _mm_bias_body,
        grid=(N_EDGES // BE,),
        in_specs=[
            pl.BlockSpec((BE, D), lambda i: (i, 0)),
            pl.BlockSpec((D, D), lambda i: (0, 0)),
            pl.BlockSpec((1, D), lambda i: (0, 0)),
        ],
        out_specs=pl.BlockSpec((BE, D), lambda i: (i, 0)),
        out_shape=jax.ShapeDtypeStruct((N_EDGES, D), jnp.float32),
    )(edge_attr, lin_Wt, lin_b.reshape(1, D))


def _combine_body(x_ref, p0_ref, p1_ref, w_ref, b_ref, o_ref):
    h = x_ref[...] + p0_ref[...] + p1_ref[...]
    o_ref[...] = (
        jnp.dot(h, w_ref[...], preferred_element_type=jnp.float32) + b_ref[...]
    )


def _combine(x, p0, p1, nn_Wt, nn_b):
    return pl.pallas_call(
        _combine_body,
        grid=(N_NODES // BN,),
        in_specs=[
            pl.BlockSpec((BN, D), lambda i: (i, 0)),
            pl.BlockSpec((BN, D), lambda i: (i, 0)),
            pl.BlockSpec((BN, D), lambda i: (i, 0)),
            pl.BlockSpec((D, D), lambda i: (0, 0)),
            pl.BlockSpec((1, D), lambda i: (0, 0)),
        ],
        out_specs=pl.BlockSpec((BN, D), lambda i: (i, 0)),
        out_shape=jax.ShapeDtypeStruct((N_NODES, D), jnp.float32),
    )(x, p0, p1, nn_Wt, nn_b.reshape(1, D))


_sc_mesh = plsc.VectorSubcoreMesh(core_axis_name="c", subcore_axis_name="s")


@functools.partial(
    pl.kernel,
    out_type=jax.ShapeDtypeStruct((NC * N_PAD, D), jnp.float32),
    mesh=_sc_mesh,
    scratch_types=[
        pltpu.VMEM((CHUNK,), jnp.int32),
        pltpu.VMEM((CHUNK,), jnp.int32),
        pltpu.VMEM((CHUNK,), jnp.int32),
        pltpu.VMEM((CHUNK,), jnp.int32),
        pltpu.VMEM((CHUNK,), jnp.int32),
        pltpu.VMEM((CHUNK,), jnp.int32),
        pltpu.VMEM((CHUNK,), jnp.int32),
        pltpu.VMEM((CHUNK,), jnp.int32),
        pltpu.VMEM((CHUNK, D), jnp.float32),
        pltpu.VMEM((CHUNK, D), jnp.float32),
        pltpu.VMEM((CHUNK, D), jnp.float32),
        pltpu.VMEM((CHUNK, D), jnp.float32),
        pltpu.VMEM((ZCHUNK, D), jnp.float32),
        pltpu.VMEM_SHARED((N_PAD, D), jnp.float32),
        pltpu.SemaphoreType.DMA,
        pltpu.SemaphoreType.DMA,
        pltpu.SemaphoreType.DMA,
        pltpu.SemaphoreType.DMA,
        pltpu.SemaphoreType.DMA,
        pltpu.SemaphoreType.DMA,
    ],
)
def _sc_aggregate(src_hbm, dst_hbm, x_hbm, eproj_hbm, out_hbm,
                  s0, s1, s2, s3, d0, d1, d2, d3,
                  xr0, xr1, er0, er1, zbuf_v, acc_sh,
                  isem0, isem1, isem2, isem3, insem0, insem1):
    cid = lax.axis_index("c")
    sid = lax.axis_index("s")
    wid = sid * NC + cid
    srcb = (s0, s1, s2, s3)
    dstb = (d0, d1, d2, d3)
    xrb = (xr0, xr1)
    erb = (er0, er1)
    isems = (isem0, isem1, isem2, isem3)
    insems = (insem0, insem1)

    zero = jnp.zeros((16,), jnp.float32)

    def zrow(r, carry):
        for g in range(D // 16):
            zbuf_v[r, pl.ds(g * 16, 16)] = zero
        return carry

    lax.fori_loop(0, ZCHUNK, zrow, 0)
    for z in range(ROWS_PER_TILE // ZCHUNK):
        ro = sid * ROWS_PER_TILE + z * ZCHUNK
        pltpu.sync_copy(zbuf_v, acc_sh.at[pl.ds(ro, ZCHUNK)])
    plsc.subcore_barrier()

    def ebase(c):
        return pl.multiple_of(wid * E_PER_W + c * CHUNK, 8)

    def issue_idx(c, q):
        pltpu.async_copy(src_hbm.at[pl.ds(ebase(c), CHUNK)], srcb[q],
                         isems[q])
        pltpu.async_copy(dst_hbm.at[pl.ds(ebase(c), CHUNK)], dstb[q],
                         isems[q])

    def issue_in(c, q, b):
        pltpu.make_async_copy(src_hbm.at[pl.ds(ebase(c), CHUNK)], srcb[q],
                              isems[q]).wait()
        pltpu.make_async_copy(dst_hbm.at[pl.ds(ebase(c), CHUNK)], dstb[q],
                              isems[q]).wait()
        pltpu.async_copy(eproj_hbm.at[pl.ds(ebase(c), CHUNK)], erb[b],
                         insems[b])

    def process(c, q, b, last):
        pltpu.make_async_copy(eproj_hbm.at[pl.ds(ebase(c), CHUNK)],
                              erb[b], insems[b]).wait()
        pltpu.async_copy(x_hbm.at[srcb[q]], erb[b], insems[b],
                         add=True).wait()

        if not last:
            @pl.when(c + 2 < N_CHUNKS)
            def _():
                issue_idx(c + 2, (q + 2) % 4)

        er = erb[b]

        @plsc.parallel_loop(0, CHUNK, step=1)
        def row_body(r):
            for g in range(D // 16):
                sl = pl.ds(g * 16, 16)
                er[r, sl] = jnp.maximum(er[r, sl], 0.0)

        pltpu.sync_copy(er, acc_sh.at[dstb[q]], add=True)

        if not last:
            @pl.when(c + 2 < N_CHUNKS)
            def _():
                issue_in(c + 2, (q + 2) % 4, b)

    issue_idx(0, 0)
    issue_idx(1, 1)
    issue_in(0, 0, 0)
    issue_in(1, 1, 1)

    def quad_body(k, carry):
        for b4 in range(4):
            c = 4 * k + b4
            process(c, b4, b4 % 2, last=False)
        return carry

    lax.fori_loop(0, N_CHUNKS // 4, quad_body, 0)
    for c in range(4 * (N_CHUNKS // 4), N_CHUNKS):
        process(c, c % 4, c % 2, last=True)
    plsc.subcore_barrier()

    for z in range(ROWS_PER_TILE // ZCHUNK):
        ro = sid * ROWS_PER_TILE + z * ZCHUNK
        pltpu.sync_copy(acc_sh.at[pl.ds(ro, ZCHUNK)], zbuf_v)
        pltpu.sync_copy(
            zbuf_v, out_hbm.at[pl.ds(cid * N_PAD + ro, ZCHUNK)])


def kernel(x, edge_index, edge_attr, lin_W, lin_b, nn_W, nn_b):
    src = edge_index[0].astype(jnp.int32)
    dst = edge_index[1].astype(jnp.int32)
    e_proj = _edge_proj(edge_attr, lin_W.T, lin_b)
    parts = _sc_aggregate(src, dst, x, e_proj)
    return _combine(x, parts[:N_NODES], parts[N_PAD:N_PAD + N_NODES],
                    nn_W.T, nn_b)


# speedup vs baseline: 1.3731x; 1.3731x over previous
import functools

import jax
import jax.numpy as jnp
from jax import lax
from jax.experimental import pallas as pl
from jax.experimental.pallas import tpu as pltpu
from jax.experimental.pallas import tpu_sc as plsc

N_NODES = 10000
N_EDGES = 320000
D = 128

NC = 2
NS = 16
NW = NC * NS
E_PER_W = N_EDGES // NW
CHUNK = 40
N_CHUNKS = E_PER_W // CHUNK
N_PAD = 10240
ROWS_PER_TILE = N_PAD // NS
ZCHUNK = 64

BE = 2560
BN = 2000


def _mm_bias_body(a_ref, w_ref, b_ref, o_ref):
    o_ref[...] = (
        jnp.dot(a_ref[...], w_ref[...], preferred_element_type=jnp.float32)
        + b_ref[...]
    )


def _edge_proj(edge_attr, lin_Wt, lin_b):
    return pl.pallas_call(
        _mm_bias_body,
        grid=(N_EDGES // BE,),
        in_specs=[
            pl.BlockSpec((BE, D), lambda i: (i, 0)),
            pl.BlockSpec((D, D), lambda i: (0, 0)),
            pl.BlockSpec((1, D), lambda i: (0, 0)),
        ],
        out_specs=pl.BlockSpec((BE, D), lambda i: (i, 0)),
        out_shape=jax.ShapeDtypeStruct((N_EDGES, D), jnp.float32),
    )(edge_attr, lin_Wt, lin_b.reshape(1, D))


def _combine_body(x_ref, p0_ref, p1_ref, w_ref, b_ref, o_ref):
    h = x_ref[...] + p0_ref[...] + p1_ref[...]
    o_ref[...] = (
        jnp.dot(h, w_ref[...], preferred_element_type=jnp.float32) + b_ref[...]
    )


def _combine(x, p0, p1, nn_Wt, nn_b):
    return pl.pallas_call(
        _combine_body,
        grid=(N_NODES // BN,),
        in_specs=[
            pl.BlockSpec((BN, D), lambda i: (i, 0)),
            pl.BlockSpec((BN, D), lambda i: (i, 0)),
            pl.BlockSpec((BN, D), lambda i: (i, 0)),
            pl.BlockSpec((D, D), lambda i: (0, 0)),
            pl.BlockSpec((1, D), lambda i: (0, 0)),
        ],
        out_specs=pl.BlockSpec((BN, D), lambda i: (i, 0)),
        out_shape=jax.ShapeDtypeStruct((N_NODES, D), jnp.float32),
    )(x, p0, p1, nn_Wt, nn_b.reshape(1, D))


def _run_if(cond):
    def deco(f):
        if cond:
            f()
    return deco


_sc_mesh = plsc.VectorSubcoreMesh(core_axis_name="c", subcore_axis_name="s")


@functools.partial(
    pl.kernel,
    out_type=jax.ShapeDtypeStruct((NC * N_PAD, D), jnp.float32),
    mesh=_sc_mesh,
    scratch_types=[
        pltpu.VMEM((CHUNK,), jnp.int32),
        pltpu.VMEM((CHUNK,), jnp.int32),
        pltpu.VMEM((CHUNK,), jnp.int32),
        pltpu.VMEM((CHUNK,), jnp.int32),
        pltpu.VMEM((CHUNK,), jnp.int32),
        pltpu.VMEM((CHUNK,), jnp.int32),
        pltpu.VMEM((CHUNK,), jnp.int32),
        pltpu.VMEM((CHUNK,), jnp.int32),
        pltpu.VMEM((CHUNK, D), jnp.float32),
        pltpu.VMEM((CHUNK, D), jnp.float32),
        pltpu.VMEM((CHUNK, D), jnp.float32),
        pltpu.VMEM((CHUNK, D), jnp.float32),
        pltpu.VMEM((ZCHUNK, D), jnp.float32),
        pltpu.VMEM_SHARED((N_PAD, D), jnp.float32),
    ] + [pltpu.SemaphoreType.DMA] * 16,
)
def _sc_aggregate(src_hbm, dst_hbm, x_hbm, eproj_hbm, out_hbm,
                  s0, s1, s2, s3, d0, d1, d2, d3,
                  er0, er1, er2, er3, zbuf_v, acc_sh,
                  *sems):
    cid = lax.axis_index("c")
    sid = lax.axis_index("s")
    wid = sid * NC + cid
    srcb = (s0, s1, s2, s3)
    dstb = (d0, d1, d2, d3)
    erb = (er0, er1, er2, er3)
    isems = sems[0:4]
    epsems = sems[4:8]
    gasems = sems[8:12]
    scsems = sems[12:16]

    zero = jnp.zeros((16,), jnp.float32)

    def zrow(r, carry):
        for g in range(D // 16):
            zbuf_v[r, pl.ds(g * 16, 16)] = zero
        return carry

    lax.fori_loop(0, ZCHUNK, zrow, 0)
    for z in range(ROWS_PER_TILE // ZCHUNK):
        ro = sid * ROWS_PER_TILE + z * ZCHUNK
        pltpu.sync_copy(zbuf_v, acc_sh.at[pl.ds(ro, ZCHUNK)])
    plsc.subcore_barrier()

    def ebase(c):
        return pl.multiple_of(wid * E_PER_W + c * CHUNK, 8)

    def issue_idx(c, q):
        pltpu.async_copy(src_hbm.at[pl.ds(ebase(c), CHUNK)], srcb[q],
                         isems[q])
        pltpu.async_copy(dst_hbm.at[pl.ds(ebase(c), CHUNK)], dstb[q],
                         isems[q])

    def wait_idx(c, q):
        pltpu.make_async_copy(src_hbm.at[pl.ds(ebase(c), CHUNK)], srcb[q],
                              isems[q]).wait()
        pltpu.make_async_copy(dst_hbm.at[pl.ds(ebase(c), CHUNK)], dstb[q],
                              isems[q]).wait()

    def issue_eproj(c, q):
        pltpu.async_copy(eproj_hbm.at[pl.ds(ebase(c), CHUNK)], erb[q],
                         epsems[q])

    def issue_ga(c, q):
        pltpu.make_async_copy(eproj_hbm.at[pl.ds(ebase(c), CHUNK)], erb[q],
                              epsems[q]).wait()
        wait_idx(c, q)
        pltpu.async_copy(x_hbm.at[srcb[q]], erb[q], gasems[q], add=True)

    def process(c, p, traced):
        def when(cond):
            if traced:
                return pl.when(cond)
            return _run_if(bool(cond))

        @when(c + 2 < N_CHUNKS)
        def _():
            issue_idx(c + 2, (p + 2) % 4)
            issue_eproj(c + 2, (p + 2) % 4)

        @when(c + 1 < N_CHUNKS)
        def _():
            issue_ga(c + 1, (p + 1) % 4)

        pltpu.make_async_copy(x_hbm.at[srcb[p]], erb[p], gasems[p]).wait()
        er = erb[p]

        @plsc.parallel_loop(0, CHUNK, step=1)
        def row_body(r):
            for g in range(D // 16):
                sl = pl.ds(g * 16, 16)
                er[r, sl] = jnp.maximum(er[r, sl], 0.0)

        pltpu.sync_copy(er, acc_sh.at[dstb[p]], add=True)

    issue_idx(0, 0)
    issue_idx(1, 1)
    issue_eproj(0, 0)
    issue_eproj(1, 1)
    issue_ga(0, 0)

    def quad_body(k, carry):
        for b4 in range(4):
            process(4 * k + b4, b4, traced=True)
        return carry

    lax.fori_loop(0, N_CHUNKS // 4, quad_body, 0)
    for c in range(4 * (N_CHUNKS // 4), N_CHUNKS):
        process(c, c % 4, traced=False)
    plsc.subcore_barrier()

    for z in range(ROWS_PER_TILE // ZCHUNK):
        ro = sid * ROWS_PER_TILE + z * ZCHUNK
        pltpu.sync_copy(acc_sh.at[pl.ds(ro, ZCHUNK)], zbuf_v)
        pltpu.sync_copy(
            zbuf_v, out_hbm.at[pl.ds(cid * N_PAD + ro, ZCHUNK)])


def kernel(x, edge_index, edge_attr, lin_W, lin_b, nn_W, nn_b):
    src = edge_index[0].astype(jnp.int32)
    dst = edge_index[1].astype(jnp.int32)
    e_proj = _edge_proj(edge_attr, lin_W.T, lin_b)
    parts = _sc_aggregate(src, dst, x, e_proj)
    return _combine(x, parts[:N_NODES], parts[N_PAD:N_PAD + N_NODES],
                    nn_W.T, nn_b)
